# initial kernel scaffold (unmeasured)
import jax
import jax.numpy as jnp
from jax import lax
from jax.experimental import pallas as pl
from jax.experimental.pallas import tpu as pltpu

NDEV = 32
M, N = 4096, 8192
CH = M // NDEV
NSTEPS = 2 * (NDEV - 1)


def _silu_f32(v_bf16):
    y = v_bf16.astype(jnp.float32)
    return y * (1.0 / (1.0 + jnp.exp(-y)))


def _ring_ar_silu(partial):
    def body(p_ref, o_ref, comm, acc, lbuf, obuf, send_sems, recv_sems,
             lsem, osem):
        my = lax.axis_index("i")
        left = lax.rem(my + NDEV - 1, NDEV)
        right = lax.rem(my + 1, NDEV)

        barrier = pltpu.get_barrier_semaphore()
        for nbr in (left, right):
            pl.semaphore_signal(barrier, inc=1, device_id=(nbr,),
                                device_id_type=pl.DeviceIdType.MESH)
        pl.semaphore_wait(barrier, 2)

        cp = pltpu.make_async_copy(p_ref.at[pl.ds(my * CH, CH), :], acc, lsem)
        cp.start()
        cp.wait()

        for s in range(NDEV - 1):
            slot = s % 2
            c_next = lax.rem(my - (s + 1) + 2 * NDEV, NDEV)
            lcp = pltpu.make_async_copy(
                p_ref.at[pl.ds(c_next * CH, CH), :], lbuf, lsem)
            lcp.start()
            rdma = pltpu.make_async_remote_copy(
                src_ref=acc,
                dst_ref=comm.at[slot],
                send_sem=send_sems.at[s],
                recv_sem=recv_sems.at[s],
                device_id=(right,),
                device_id_type=pl.DeviceIdType.MESH,
            )
            rdma.start()
            rdma.wait()
            lcp.wait()
            acc[...] = comm[slot] + lbuf[...]


        def store(chunk_idx, vals_bf16):
            obuf[...] = _silu_f32(vals_bf16)
            ocp = pltpu.make_async_copy(
                obuf, o_ref.at[pl.ds(chunk_idx * CH, CH), :], osem)
            ocp.start()
            ocp.wait()

        own = lax.rem(my + 1, NDEV)
        for t in range(NDEV - 1):
            u = (NDEV - 1) + t
            slot = u % 2
            src = acc if t == 0 else comm.at[(u - 1) % 2]
            rdma = pltpu.make_async_remote_copy(
                src_ref=src,
                dst_ref=comm.at[slot],
                send_sem=send_sems.at[u],
                recv_sem=recv_sems.at[u],
                device_id=(right,),
                device_id_type=pl.DeviceIdType.MESH,
            )
            rdma.start()
            if t == 0:
                store(own, acc[...])
            rdma.wait()
            g = lax.rem(my - t + 2 * NDEV, NDEV)
            store(g, comm[slot])

    return pl.pallas_call(
        body,
        out_shape=jax.ShapeDtypeStruct((M, N), jnp.float32),
        in_specs=[pl.BlockSpec(memory_space=pltpu.MemorySpace.ANY)],
        out_specs=pl.BlockSpec(memory_space=pltpu.MemorySpace.ANY),
        scratch_shapes=[
            pltpu.VMEM((2, CH, N), jnp.bfloat16),
            pltpu.VMEM((CH, N), jnp.bfloat16),
            pltpu.VMEM((CH, N), jnp.bfloat16),
            pltpu.VMEM((CH, N), jnp.float32),
            pltpu.SemaphoreType.DMA((NSTEPS,)),
            pltpu.SemaphoreType.DMA((NSTEPS,)),
            pltpu.SemaphoreType.DMA,
            pltpu.SemaphoreType.DMA,
        ],
        compiler_params=pltpu.CompilerParams(collective_id=0),
    )(partial)


def kernel(x, w_mat):
    partial = jnp.dot(x, w_mat, preferred_element_type=jnp.float32)
    return _ring_ar_silu(partial.astype(jnp.bfloat16))


# baseline (device time: 1734163 ns/iter reference)
import jax
import jax.numpy as jnp
from jax import lax
from jax.experimental import pallas as pl
from jax.experimental.pallas import tpu as pltpu

NDEV = 32
M, N = 4096, 8192
CH = M // NDEV
NSTEPS = 2 * (NDEV - 1)


def _silu_f32(v_bf16):
    y = v_bf16.astype(jnp.float32)
    return y * (1.0 / (1.0 + jnp.exp(-y)))


def _ring_ar_silu(partial):
    def body(p_ref, o_ref, comm, acc, lbuf, obuf, send_sems, recv_sems,
             lsem, osem):
        my = lax.axis_index("i")
        left = lax.rem(my + NDEV - 1, NDEV)
        right = lax.rem(my + 1, NDEV)

        barrier = pltpu.get_barrier_semaphore()
        for nbr in (left, right):
            pl.semaphore_signal(barrier, inc=1, device_id=(nbr,),
                                device_id_type=pl.DeviceIdType.MESH)
        pl.semaphore_wait(barrier, 2)

        cp = pltpu.make_async_copy(p_ref.at[pl.ds(my * CH, CH), :], acc, lsem)
        cp.start()
        cp.wait()

        for s in range(NDEV - 1):
            slot = s % 2
            c_next = lax.rem(my - (s + 1) + 2 * NDEV, NDEV)
            lcp = pltpu.make_async_copy(
                p_ref.at[pl.ds(c_next * CH, CH), :], lbuf, lsem)
            lcp.start()
            rdma = pltpu.make_async_remote_copy(
                src_ref=acc,
                dst_ref=comm.at[slot],
                send_sem=send_sems.at[s],
                recv_sem=recv_sems.at[s],
                device_id=(right,),
                device_id_type=pl.DeviceIdType.MESH,
            )
            rdma.start()
            rdma.wait()
            lcp.wait()
            acc[...] = comm[slot] + lbuf[...]


        def store(chunk_idx, vals_bf16):
            obuf[...] = _silu_f32(vals_bf16)
            ocp = pltpu.make_async_copy(
                obuf, o_ref.at[pl.ds(chunk_idx * CH, CH), :], osem)
            ocp.start()
            ocp.wait()

        own = lax.rem(my + 1, NDEV)
        for t in range(NDEV - 1):
            u = (NDEV - 1) + t
            slot = u % 2
            src = acc if t == 0 else comm.at[(u - 1) % 2]
            rdma = pltpu.make_async_remote_copy(
                src_ref=src,
                dst_ref=comm.at[slot],
                send_sem=send_sems.at[u],
                recv_sem=recv_sems.at[u],
                device_id=(right,),
                device_id_type=pl.DeviceIdType.MESH,
            )
            rdma.start()
            if t == 0:
                store(own, acc[...])
            rdma.wait()
            g = lax.rem(my - t + 2 * NDEV, NDEV)
            store(g, comm[slot])

    return pl.pallas_call(
        body,
        out_shape=jax.ShapeDtypeStruct((M, N), jnp.float32),
        in_specs=[pl.BlockSpec(memory_space=pltpu.MemorySpace.HBM)],
        out_specs=pl.BlockSpec(memory_space=pltpu.MemorySpace.HBM),
        scratch_shapes=[
            pltpu.VMEM((2, CH, N), jnp.bfloat16),
            pltpu.VMEM((CH, N), jnp.bfloat16),
            pltpu.VMEM((CH, N), jnp.bfloat16),
            pltpu.VMEM((CH, N), jnp.float32),
            pltpu.SemaphoreType.DMA((NSTEPS,)),
            pltpu.SemaphoreType.DMA((NSTEPS,)),
            pltpu.SemaphoreType.DMA,
            pltpu.SemaphoreType.DMA,
        ],
        compiler_params=pltpu.CompilerParams(collective_id=0),
    )(partial)


def kernel(x, w_mat):
    partial = jnp.dot(x, w_mat, preferred_element_type=jnp.float32)
    return _ring_ar_silu(partial.astype(jnp.bfloat16))
